# baseline (device time: 55466 ns/iter reference)
import jax
import jax.numpy as jnp
from jax import lax
from jax.experimental import pallas as pl
from jax.experimental.pallas import tpu as pltpu

N_DEV = 8
N_STEPS = 3


def kernel(x, Win0, Wout0, Win1, Wout1, Win2, Wout2):
    b, d = x.shape
    rows_out = b // N_DEV

    def body(x_ref, win0_ref, wout0_ref, win1_ref, wout1_ref,
             win2_ref, wout2_ref, out_ref,
             acc_ref, comm_ref, send_sems, recv_sems):
        my = lax.axis_index("i")

        barrier_sem = pltpu.get_barrier_semaphore()
        for s in range(N_STEPS):
            pl.semaphore_signal(
                barrier_sem, inc=1,
                device_id=(my ^ (1 << s),),
                device_id_type=pl.DeviceIdType.MESH,
            )
        pl.semaphore_wait(barrier_sem, N_STEPS)

        xv = x_ref[...]
        layers = [(win0_ref, wout0_ref), (win1_ref, wout1_ref),
                  (win2_ref, wout2_ref)]
        for layer, (win_ref, wout_ref) in enumerate(layers):
            h = jnp.maximum(
                jnp.dot(xv, win_ref[...], preferred_element_type=jnp.float32),
                0.0,
            )
            acc_ref[...] = jnp.dot(
                h, wout_ref[...], preferred_element_type=jnp.float32
            )
            for s in range(N_STEPS):
                slot = layer * N_STEPS + s
                partner = my ^ (1 << s)
                rdma = pltpu.make_async_remote_copy(
                    src_ref=acc_ref,
                    dst_ref=comm_ref.at[slot],
                    send_sem=send_sems.at[slot],
                    recv_sem=recv_sems.at[slot],
                    device_id=(partner,),
                    device_id_type=pl.DeviceIdType.MESH,
                )
                rdma.start()
                rdma.wait()
                acc_ref[...] += comm_ref[slot]
            xv = acc_ref[...]

        out_ref[...] = acc_ref[pl.ds(my * rows_out, rows_out), :]

    return pl.pallas_call(
        body,
        out_shape=jax.ShapeDtypeStruct((rows_out, d), jnp.float32),
        in_specs=[pl.BlockSpec(memory_space=pltpu.VMEM)] * 7,
        out_specs=pl.BlockSpec(memory_space=pltpu.VMEM),
        scratch_shapes=[
            pltpu.VMEM((b, d), jnp.float32),
            pltpu.VMEM((N_STEPS * 3, b, d), jnp.float32),
            pltpu.SemaphoreType.DMA((N_STEPS * 3,)),
            pltpu.SemaphoreType.DMA((N_STEPS * 3,)),
        ],
        compiler_params=pltpu.CompilerParams(collective_id=0),
    )(x, Win0, Wout0, Win1, Wout1, Win2, Wout2)


# device time: 41287 ns/iter; 1.3434x vs baseline; 1.3434x over previous
import jax
import jax.numpy as jnp
from jax import lax
from jax.experimental import pallas as pl
from jax.experimental.pallas import tpu as pltpu

N_DEV = 8
N_STEPS = 3
N_LAYERS = 3
ORDERS = ((1, 3, 4), (3, 4, 1))
N_HALVES = 2


def kernel(x, Win0, Wout0, Win1, Wout1, Win2, Wout2):
    b, d = x.shape
    rows_out = b // N_DEV
    half = b // N_HALVES
    n_slots = N_LAYERS * N_STEPS * N_HALVES

    def body(x_ref, win0_ref, wout0_ref, win1_ref, wout1_ref,
             win2_ref, wout2_ref, out_ref,
             acc_ref, comm_ref, send_sems, recv_sems):
        my = lax.axis_index("i")

        barrier_sem = pltpu.get_barrier_semaphore()
        for m in (1, 3, 4):
            pl.semaphore_signal(
                barrier_sem, inc=1,
                device_id=(my ^ m,),
                device_id_type=pl.DeviceIdType.MESH,
            )
        pl.semaphore_wait(barrier_sem, 3)

        xv = x_ref[...]
        layers = [(win0_ref, wout0_ref), (win1_ref, wout1_ref),
                  (win2_ref, wout2_ref)]
        for layer, (win_ref, wout_ref) in enumerate(layers):
            h = jnp.maximum(
                jnp.dot(xv, win_ref[...], preferred_element_type=jnp.float32),
                0.0,
            )
            acc_ref[...] = jnp.dot(
                h, wout_ref[...], preferred_element_type=jnp.float32
            )
            for s in range(N_STEPS):
                rdmas = []
                for hf in range(N_HALVES):
                    slot = (layer * N_STEPS + s) * N_HALVES + hf
                    rdma = pltpu.make_async_remote_copy(
                        src_ref=acc_ref.at[pl.ds(hf * half, half), :],
                        dst_ref=comm_ref.at[slot],
                        send_sem=send_sems.at[slot],
                        recv_sem=recv_sems.at[slot],
                        device_id=(my ^ ORDERS[hf][s],),
                        device_id_type=pl.DeviceIdType.MESH,
                    )
                    rdma.start()
                    rdmas.append((hf, slot, rdma))
                for hf, slot, rdma in rdmas:
                    rdma.wait()
                    acc_ref[pl.ds(hf * half, half), :] += comm_ref[slot]
            xv = acc_ref[...]

        out_ref[...] = acc_ref[pl.ds(my * rows_out, rows_out), :]

    return pl.pallas_call(
        body,
        out_shape=jax.ShapeDtypeStruct((rows_out, d), jnp.float32),
        in_specs=[pl.BlockSpec(memory_space=pltpu.VMEM)] * 7,
        out_specs=pl.BlockSpec(memory_space=pltpu.VMEM),
        scratch_shapes=[
            pltpu.VMEM((b, d), jnp.float32),
            pltpu.VMEM((n_slots, half, d), jnp.float32),
            pltpu.SemaphoreType.DMA((n_slots,)),
            pltpu.SemaphoreType.DMA((n_slots,)),
        ],
        compiler_params=pltpu.CompilerParams(collective_id=0),
    )(x, Win0, Wout0, Win1, Wout1, Win2, Wout2)


# device time: 36130 ns/iter; 1.5352x vs baseline; 1.1427x over previous
import jax
import jax.numpy as jnp
from jax import lax
from jax.experimental import pallas as pl
from jax.experimental.pallas import tpu as pltpu

N_DEV = 8
N_STEPS = 3
N_LAYERS = 3
ORDERS = ((1, 3, 4), (3, 4, 1))
N_HALVES = 2


def kernel(x, Win0, Wout0, Win1, Wout1, Win2, Wout2):
    b, d = x.shape
    rows_out = b // N_DEV
    half = b // N_HALVES
    n_bfly = 2 * N_STEPS * N_HALVES
    n_a2a = N_DEV - 1

    def body(x_ref, win0_ref, wout0_ref, win1_ref, wout1_ref,
             win2_ref, wout2_ref, out_ref,
             acc_ref, comm_ref, stage_ref, a2a_ref,
             send_sems, recv_sems, a2a_send_sems, a2a_recv_sems):
        my = lax.axis_index("i")

        barrier_sem = pltpu.get_barrier_semaphore()
        for m in (1, 3, 4):
            pl.semaphore_signal(
                barrier_sem, inc=1,
                device_id=(my ^ m,),
                device_id_type=pl.DeviceIdType.MESH,
            )
        pl.semaphore_wait(barrier_sem, 3)

        layers = [(win0_ref, wout0_ref), (win1_ref, wout1_ref),
                  (win2_ref, wout2_ref)]

        def partial_half(layer, xh):
            win_ref, wout_ref = layers[layer]
            hh = jnp.maximum(
                jnp.dot(xh, win_ref[...], preferred_element_type=jnp.float32),
                0.0,
            )
            return jnp.dot(hh, wout_ref[...],
                           preferred_element_type=jnp.float32)

        def start_bfly(layer, s, hf):
            slot = (layer * N_STEPS + s) * N_HALVES + hf
            rdma = pltpu.make_async_remote_copy(
                src_ref=acc_ref.at[pl.ds(hf * half, half), :],
                dst_ref=comm_ref.at[slot],
                send_sem=send_sems.at[slot],
                recv_sem=recv_sems.at[slot],
                device_id=(my ^ ORDERS[hf][s],),
                device_id_type=pl.DeviceIdType.MESH,
            )
            rdma.start()
            return slot, rdma

        inflight = {}
        for hf in range(N_HALVES):
            acc_ref[pl.ds(hf * half, half), :] = partial_half(
                0, x_ref[pl.ds(hf * half, half), :]
            )
            inflight[hf] = start_bfly(0, 0, hf)

        for layer in range(2):
            for s in range(N_STEPS):
                for hf in range(N_HALVES):
                    slot, rdma = inflight[hf]
                    rdma.wait()
                    acc_ref[pl.ds(hf * half, half), :] += comm_ref[slot]
                    if s < N_STEPS - 1:
                        inflight[hf] = start_bfly(layer, s + 1, hf)
                    else:
                        nxt = partial_half(
                            layer + 1, acc_ref[pl.ds(hf * half, half), :]
                        )
                        acc_ref[pl.ds(hf * half, half), :] = nxt
                        if layer < 1:
                            inflight[hf] = start_bfly(layer + 1, 0, hf)

        a2a = []
        for o in range(1, N_DEV):
            tgt = my ^ o
            stage_ref[o - 1] = acc_ref[pl.ds(tgt * rows_out, rows_out), :]
            rdma = pltpu.make_async_remote_copy(
                src_ref=stage_ref.at[o - 1],
                dst_ref=a2a_ref.at[o - 1],
                send_sem=a2a_send_sems.at[o - 1],
                recv_sem=a2a_recv_sems.at[o - 1],
                device_id=(tgt,),
                device_id_type=pl.DeviceIdType.MESH,
            )
            rdma.start()
            a2a.append(rdma)
        for rdma in a2a:
            rdma.wait()
        total = acc_ref[pl.ds(my * rows_out, rows_out), :]
        for o in range(1, N_DEV):
            total += a2a_ref[o - 1]
        out_ref[...] = total

    return pl.pallas_call(
        body,
        out_shape=jax.ShapeDtypeStruct((rows_out, d), jnp.float32),
        in_specs=[pl.BlockSpec(memory_space=pltpu.VMEM)] * 7,
        out_specs=pl.BlockSpec(memory_space=pltpu.VMEM),
        scratch_shapes=[
            pltpu.VMEM((b, d), jnp.float32),
            pltpu.VMEM((n_bfly, half, d), jnp.float32),
            pltpu.VMEM((n_a2a, rows_out, d), jnp.float32),
            pltpu.VMEM((n_a2a, rows_out, d), jnp.float32),
            pltpu.SemaphoreType.DMA((n_bfly,)),
            pltpu.SemaphoreType.DMA((n_bfly,)),
            pltpu.SemaphoreType.DMA((n_a2a,)),
            pltpu.SemaphoreType.DMA((n_a2a,)),
        ],
        compiler_params=pltpu.CompilerParams(collective_id=0),
    )(x, Win0, Wout0, Win1, Wout1, Win2, Wout2)


# device time: 30299 ns/iter; 1.8306x vs baseline; 1.1924x over previous
import jax
import jax.numpy as jnp
from jax import lax
from jax.experimental import pallas as pl
from jax.experimental.pallas import tpu as pltpu

N_DEV = 8
N_STEPS = 3
ORDERS = ((1, 3, 4), (3, 4, 1))
N_HALVES = 2


def kernel(x, Win0, Wout0, Win1, Wout1, Win2, Wout2):
    b, d = x.shape
    rows_out = b // N_DEV
    half = b // N_HALVES
    n_bfly = 2 * N_STEPS * N_HALVES
    n_a2a = N_DEV - 1

    bf16 = jnp.bfloat16

    def body(x_ref, win0_ref, wout0_ref, win1_ref, wout1_ref,
             win2_ref, wout2_ref, out_ref,
             acc_ref, sendbuf_ref, comm_ref, stage_ref, a2a_ref,
             send_sems, recv_sems, a2a_send_sems, a2a_recv_sems):
        my = lax.axis_index("i")

        barrier_sem = pltpu.get_barrier_semaphore()
        for m in (1, 3, 4):
            pl.semaphore_signal(
                barrier_sem, inc=1,
                device_id=(my ^ m,),
                device_id_type=pl.DeviceIdType.MESH,
            )
        pl.semaphore_wait(barrier_sem, 3)

        layers = [(win0_ref, wout0_ref), (win1_ref, wout1_ref),
                  (win2_ref, wout2_ref)]

        def partial_half(layer, xh_bf16):
            win_ref, wout_ref = layers[layer]
            hh = jnp.maximum(
                jnp.dot(xh_bf16, win_ref[...],
                        preferred_element_type=jnp.float32),
                0.0,
            ).astype(bf16)
            return jnp.dot(hh, wout_ref[...],
                           preferred_element_type=jnp.float32)

        def start_bfly(layer, s, hf):
            slot = (layer * N_STEPS + s) * N_HALVES + hf
            sendbuf_ref[slot] = acc_ref[pl.ds(hf * half, half), :].astype(bf16)
            rdma = pltpu.make_async_remote_copy(
                src_ref=sendbuf_ref.at[slot],
                dst_ref=comm_ref.at[slot],
                send_sem=send_sems.at[slot],
                recv_sem=recv_sems.at[slot],
                device_id=(my ^ ORDERS[hf][s],),
                device_id_type=pl.DeviceIdType.MESH,
            )
            rdma.start()
            return slot, rdma

        inflight = {}
        for hf in range(N_HALVES):
            acc_ref[pl.ds(hf * half, half), :] = partial_half(
                0, x_ref[pl.ds(hf * half, half), :].astype(bf16)
            )
            inflight[hf] = start_bfly(0, 0, hf)

        for layer in range(2):
            for s in range(N_STEPS):
                for hf in range(N_HALVES):
                    slot, rdma = inflight[hf]
                    rdma.wait()
                    acc_ref[pl.ds(hf * half, half), :] += (
                        comm_ref[slot].astype(jnp.float32)
                    )
                    if s < N_STEPS - 1:
                        inflight[hf] = start_bfly(layer, s + 1, hf)
                    else:
                        nxt = partial_half(
                            layer + 1,
                            acc_ref[pl.ds(hf * half, half), :].astype(bf16),
                        )
                        acc_ref[pl.ds(hf * half, half), :] = nxt
                        if layer < 1:
                            inflight[hf] = start_bfly(layer + 1, 0, hf)

        a2a = []
        for o in range(1, N_DEV):
            tgt = my ^ o
            stage_ref[o - 1] = acc_ref[
                pl.ds(tgt * rows_out, rows_out), :
            ].astype(bf16)
            rdma = pltpu.make_async_remote_copy(
                src_ref=stage_ref.at[o - 1],
                dst_ref=a2a_ref.at[o - 1],
                send_sem=a2a_send_sems.at[o - 1],
                recv_sem=a2a_recv_sems.at[o - 1],
                device_id=(tgt,),
                device_id_type=pl.DeviceIdType.MESH,
            )
            rdma.start()
            a2a.append(rdma)
        for rdma in a2a:
            rdma.wait()
        total = acc_ref[pl.ds(my * rows_out, rows_out), :]
        for o in range(1, N_DEV):
            total += a2a_ref[o - 1].astype(jnp.float32)
        out_ref[...] = total

    return pl.pallas_call(
        body,
        out_shape=jax.ShapeDtypeStruct((rows_out, d), jnp.float32),
        in_specs=[pl.BlockSpec(memory_space=pltpu.VMEM)] * 7,
        out_specs=pl.BlockSpec(memory_space=pltpu.VMEM),
        scratch_shapes=[
            pltpu.VMEM((b, d), jnp.float32),
            pltpu.VMEM((n_bfly, half, d), bf16),
            pltpu.VMEM((n_bfly, half, d), bf16),
            pltpu.VMEM((n_a2a, rows_out, d), bf16),
            pltpu.VMEM((n_a2a, rows_out, d), bf16),
            pltpu.SemaphoreType.DMA((n_bfly,)),
            pltpu.SemaphoreType.DMA((n_bfly,)),
            pltpu.SemaphoreType.DMA((n_a2a,)),
            pltpu.SemaphoreType.DMA((n_a2a,)),
        ],
        compiler_params=pltpu.CompilerParams(collective_id=0),
    )(
        x.astype(bf16),
        Win0.astype(bf16), Wout0.astype(bf16),
        Win1.astype(bf16), Wout1.astype(bf16),
        Win2.astype(bf16), Wout2.astype(bf16),
    )
